# Initial kernel scaffold; baseline (speedup 1.0000x reference)
import jax
import jax.numpy as jnp
from jax.experimental import pallas as pl


def kernel(pose_coords, pose_stack_block_coord_offset, pose_stack_block_type, pose_stack_min_bond_separation, bt_path_distance, bt_atom_is_hydrogen, bt_tile_lk_ball_params, lk_ball_global_params, water_gen_global_params, sp2_water_tors, sp3_water_tors, ring_water_tors):
    raise NotImplementedError("write your pallas kernel here")


# speedup vs baseline: 833.0724x; 833.0724x over previous
import jax
import jax.numpy as jnp
from jax.experimental import pallas as pl
from jax.experimental.pallas import tpu as pltpu

_P, _B, _A, _NBT, _W = 4, 32, 32, 24, 4
_N = _B * _A
_TI = 256
_TB = _TI // _A
_T = _N // _TI
_RX, _RY, _RZ = 0.057, 0.31, 0.95


def _water_dirs(cx, cy, cz, nx, ny, nz):
    vx, vy, vz = cx - nx, cy - ny, cz - nz
    s = 1.0 / (jnp.sqrt(vx * vx + vy * vy + vz * vz) + 1e-6)
    vx, vy, vz = vx * s, vy * s, vz * s
    p1x = vy * _RZ - vz * _RY
    p1y = vz * _RX - vx * _RZ
    p1z = vx * _RY - vy * _RX
    s1 = 1.0 / (jnp.sqrt(p1x * p1x + p1y * p1y + p1z * p1z) + 1e-6)
    p1x, p1y, p1z = p1x * s1, p1y * s1, p1z * s1
    p2x = vy * p1z - vz * p1y
    p2y = vz * p1x - vx * p1z
    p2z = vx * p1y - vy * p1x
    return p1x, p1y, p1z, p2x, p2y, p2z


def _body(crdr_ref, nbr_ref, crdc_ref, nbc_ref, tblN_ref, tblA_ref,
          pathN_ref, msbN_ref, torsc_ref, torsr_ref,
          bt_ref, wdist_ref, ramp_ref, out_ref,
          wcx_ref, wcy_ref, wcz_ref, w0x_ref, w0y_ref, w0z_ref,
          parN_ref, parA_ref):
    f32 = jnp.float32
    pp = pl.program_id(0)
    t = pl.program_id(1)
    r0 = t * _TI
    wd = wdist_ref[0, 0]

    @pl.when(t == 0)
    def _():
        jva = jax.lax.broadcasted_iota(jnp.int32, (1, _N), 1) // _A
        btN = jnp.zeros((1, _N), jnp.int32)
        for b in range(_B):
            btN = jnp.where(jva == b, bt_ref[pp, b], btN)
        for k in range(5):
            acc = jnp.zeros((1, _N), f32)
            for tt in range(_NBT):
                acc = jnp.where(btN == tt, tblN_ref[k, tt:tt + 1, :], acc)
            parN_ref[k:k + 1, :] = acc
        for b in range(_B):
            chunk = tblA_ref[pl.ds(bt_ref[pp, b], 1), :, :]
            parA_ref[b * _A:(b + 1) * _A, :] = chunk[0]
        ccx = crdc_ref[0, :, 0:1]
        ccy = crdc_ref[0, :, 1:2]
        ccz = crdc_ref[0, :, 2:3]
        ncx = nbc_ref[0, :, 0:1]
        ncy = nbc_ref[0, :, 1:2]
        ncz = nbc_ref[0, :, 2:3]
        q1x, q1y, q1z, q2x, q2y, q2z = _water_dirs(ccx, ccy, ccz, ncx, ncy, ncz)
        ctc = jnp.cos(torsc_ref[...])
        stc = jnp.sin(torsc_ref[...])
        wcx_ref[...] = ccx + wd * (q1x * ctc + q2x * stc)
        wcy_ref[...] = ccy + wd * (q1y * ctc + q2y * stc)
        wcz_ref[...] = ccz + wd * (q1z * ctc + q2z * stc)
        rcx = crdr_ref[0, 0:1, :]
        rcy = crdr_ref[0, 1:2, :]
        rcz = crdr_ref[0, 2:3, :]
        rnx = nbr_ref[0, 0:1, :]
        rny = nbr_ref[0, 1:2, :]
        rnz = nbr_ref[0, 2:3, :]
        r1x, r1y, r1z, r2x, r2y, r2z = _water_dirs(rcx, rcy, rcz, rnx, rny, rnz)
        t0 = torsr_ref[0:1, 0:1]
        ct0 = jnp.cos(t0)
        st0 = jnp.sin(t0)
        w0x_ref[...] = rcx + wd * (r1x * ct0 + r2x * st0)
        w0y_ref[...] = rcy + wd * (r1y * ct0 + r2y * st0)
        w0z_ref[...] = rcz + wd * (r1z * ct0 + r2z * st0)

    intra = jnp.concatenate(
        [pathN_ref[pl.ds(bt_ref[pp, t * _TB + lb], 1), :, :][0]
         for lb in range(_TB)], axis=0)
    inter = jnp.concatenate(
        [jnp.broadcast_to(msbN_ref[0, pl.ds(t * _TB + lb, 1), :], (_A, _N))
         for lb in range(_TB)], axis=0)
    ri = r0 + jax.lax.broadcasted_iota(jnp.int32, (_TI, _N), 0)
    cj = jax.lax.broadcasted_iota(jnp.int32, (_TI, _N), 1)
    same_blk = (ri // _A) == (cj // _A)
    sep = jnp.where(same_blk, intra, inter)
    hvy_j = parN_ref[4:5, :]
    hvy_i = parA_ref[pl.ds(r0, _TI), 4:5]
    pm = (jnp.where((sep >= 4.0) & (ri != cj), 1.0, 0.0)
          * (hvy_i * hvy_j))

    cxr = crdr_ref[0, 0:1, :]
    cyr = crdr_ref[0, 1:2, :]
    czr = crdr_ref[0, 2:3, :]
    cxi = crdc_ref[0, pl.ds(r0, _TI), 0:1]
    cyi = crdc_ref[0, pl.ds(r0, _TI), 1:2]
    czi = crdc_ref[0, pl.ds(r0, _TI), 2:3]
    dxx = cxi - cxr
    dyy = cyi - cyr
    dzz = czi - czr
    d = jnp.sqrt(jnp.maximum(dxx * dxx + dyy * dyy + dzz * dzz, 1e-6))
    dg_j = parN_ref[0:1, :]
    rad_j = parN_ref[2:3, :]
    pol_j = parN_ref[3:4, :]
    dg_i = parA_ref[pl.ds(r0, _TI), 0:1]
    lam_i = parA_ref[pl.ds(r0, _TI), 1:2]
    rad_i = parA_ref[pl.ds(r0, _TI), 2:3]
    pol_i = parA_ref[pl.ds(r0, _TI), 3:4]
    xq = (d - (rad_i + rad_j)) / lam_i
    lk_iso_p = dg_i * jnp.exp(-(xq * xq)) * pm * pol_i
    s_iso = jnp.sum(lk_iso_p)

    rr = ramp_ref[0, 0]
    inv_r2 = 1.0 / (rr * rr)
    occ = jnp.zeros((_TI, _N), f32)
    for w in range(_W):
        ax = wcx_ref[pl.ds(r0, _TI), w:w + 1] - cxr
        ay = wcy_ref[pl.ds(r0, _TI), w:w + 1] - cyr
        az = wcz_ref[pl.ds(r0, _TI), w:w + 1] - czr
        d2w = jnp.maximum(ax * ax + ay * ay + az * az, 1e-6)
        occ = occ + jnp.exp(-d2w * inv_r2)
    s_ball = jnp.sum(lk_iso_p * jnp.minimum(occ, 1.0))

    bx = wcx_ref[pl.ds(r0, _TI), 0:1] - w0x_ref[...]
    by = wcy_ref[pl.ds(r0, _TI), 0:1] - w0y_ref[...]
    bz = wcz_ref[pl.ds(r0, _TI), 0:1] - w0z_ref[...]
    d2b = jnp.maximum(bx * bx + by * by + bz * bz, 1e-6)
    bp = jnp.exp(-d2b * inv_r2) * pm * pol_i * pol_j
    s_br = jnp.sum(0.5 * (dg_i + dg_j) * bp)
    s_un = jnp.sum(bp)

    i4 = jax.lax.broadcasted_iota(jnp.int32, (1, 4), 1)
    vec = (jnp.where(i4 == 0, s_iso, 0.0) + jnp.where(i4 == 1, s_ball, 0.0)
           + jnp.where(i4 == 2, s_br, 0.0) + jnp.where(i4 == 3, s_un, 0.0))

    @pl.when(t == 0)
    def _():
        out_ref[0] = vec

    @pl.when(t > 0)
    def _():
        out_ref[0] = out_ref[0] + vec


def kernel(pose_coords, pose_stack_block_coord_offset, pose_stack_block_type,
           pose_stack_min_bond_separation, bt_path_distance, bt_atom_is_hydrogen,
           bt_tile_lk_ball_params, lk_ball_global_params, water_gen_global_params,
           sp2_water_tors, sp3_water_tors, ring_water_tors):
    f32 = jnp.float32
    crd_c = pose_coords.astype(f32)
    nb_c = jnp.roll(crd_c, 1, axis=1)
    crd_r = jnp.transpose(crd_c, (0, 2, 1))
    nb_r = jnp.transpose(nb_c, (0, 2, 1))
    prm = bt_tile_lk_ball_params[:, 0].astype(f32)
    is_h0 = (bt_atom_is_hydrogen == 0)
    tblA = jnp.stack([
        prm[..., 0] * 5.0,
        prm[..., 1] * 2.0 + 1.0,
        prm[..., 3] + 1.0,
        (((prm[..., 4] > 0.5) | (prm[..., 7] > 0.5)) & is_h0).astype(f32),
        is_h0.astype(f32),
    ], axis=-1)
    tblN = jnp.tile(jnp.transpose(tblA, (2, 0, 1)), (1, 1, _B))
    pathN = jnp.tile(bt_path_distance.astype(f32), (1, 1, _B))
    msbN = jnp.repeat(pose_stack_min_bond_separation.astype(f32),
                      _A, axis=2)
    torsv = jnp.concatenate([sp2_water_tors, sp3_water_tors]).astype(f32)
    torsc = torsv.reshape(1, _W)
    torsr = torsv.reshape(_W, 1)
    bt_s = pose_stack_block_type.astype(jnp.int32)
    wdist = water_gen_global_params[:1, :1].astype(f32)
    ramp = lk_ball_global_params[:1, :1].astype(f32)

    out = pl.pallas_call(
        _body,
        grid=(_P, _T),
        in_specs=[
            pl.BlockSpec((1, 3, _N), lambda p, t: (p, 0, 0)),
            pl.BlockSpec((1, 3, _N), lambda p, t: (p, 0, 0)),
            pl.BlockSpec((1, _N, 3), lambda p, t: (p, 0, 0)),
            pl.BlockSpec((1, _N, 3), lambda p, t: (p, 0, 0)),
            pl.BlockSpec((5, _NBT, _N), lambda p, t: (0, 0, 0)),
            pl.BlockSpec((_NBT, _A, 5), lambda p, t: (0, 0, 0)),
            pl.BlockSpec((_NBT, _A, _N), lambda p, t: (0, 0, 0)),
            pl.BlockSpec((1, _B, _N), lambda p, t: (p, 0, 0)),
            pl.BlockSpec((1, _W), lambda p, t: (0, 0)),
            pl.BlockSpec((_W, 1), lambda p, t: (0, 0)),
            pl.BlockSpec(memory_space=pltpu.SMEM),
            pl.BlockSpec(memory_space=pltpu.SMEM),
            pl.BlockSpec(memory_space=pltpu.SMEM),
        ],
        out_specs=pl.BlockSpec((1, 1, 4), lambda p, t: (p, 0, 0)),
        out_shape=jax.ShapeDtypeStruct((_P, 1, 4), f32),
        scratch_shapes=[
            pltpu.VMEM((_N, _W), f32),
            pltpu.VMEM((_N, _W), f32),
            pltpu.VMEM((_N, _W), f32),
            pltpu.VMEM((1, _N), f32),
            pltpu.VMEM((1, _N), f32),
            pltpu.VMEM((1, _N), f32),
            pltpu.VMEM((5, _N), f32),
            pltpu.VMEM((_N, 5), f32),
        ],
    )(crd_r, nb_r, crd_c, nb_c, tblN, tblA, pathN, msbN, torsc, torsr,
      bt_s, wdist, ramp)
    return out.reshape(_P, 4).T


# speedup vs baseline: 903.4965x; 1.0845x over previous
import jax
import jax.numpy as jnp
from jax.experimental import pallas as pl
from jax.experimental.pallas import tpu as pltpu

_P, _B, _A, _NBT, _W = 4, 32, 32, 24, 4
_N = _B * _A
_TI = 256
_TB = _TI // _A
_T = _N // _TI
_RX, _RY, _RZ = 0.057, 0.31, 0.95


def _water_dirs(cx, cy, cz, nx, ny, nz):
    vx, vy, vz = cx - nx, cy - ny, cz - nz
    s = 1.0 / (jnp.sqrt(vx * vx + vy * vy + vz * vz) + 1e-6)
    vx, vy, vz = vx * s, vy * s, vz * s
    p1x = vy * _RZ - vz * _RY
    p1y = vz * _RX - vx * _RZ
    p1z = vx * _RY - vy * _RX
    s1 = 1.0 / (jnp.sqrt(p1x * p1x + p1y * p1y + p1z * p1z) + 1e-6)
    p1x, p1y, p1z = p1x * s1, p1y * s1, p1z * s1
    p2x = vy * p1z - vz * p1y
    p2y = vz * p1x - vx * p1z
    p2z = vx * p1y - vy * p1x
    return p1x, p1y, p1z, p2x, p2y, p2z


def _body(crdr_ref, nbr_ref, crdc_ref, nbc_ref, tblN_ref, tblA_ref,
          pathN_ref, msbN_ref, torsc_ref, torsr_ref,
          bt_ref, wdist_ref, ramp_ref, out_ref,
          wc3_ref, w0r_ref, parN_ref, parA_ref):
    f32 = jnp.float32
    pp = pl.program_id(0)
    t = pl.program_id(1)
    r0 = t * _TI
    wd = wdist_ref[0, 0]

    @pl.when(t == 0)
    def _():
        jva = jax.lax.broadcasted_iota(jnp.int32, (1, _N), 1) // _A
        btN = jnp.zeros((1, _N), jnp.int32)
        for b in range(_B):
            btN = jnp.where(jva == b, bt_ref[pp, b], btN)
        for k in range(5):
            acc = jnp.zeros((1, _N), f32)
            for tt in range(_NBT):
                acc = jnp.where(btN == tt, tblN_ref[k, tt:tt + 1, :], acc)
            parN_ref[k:k + 1, :] = acc
        for b in range(_B):
            chunk = tblA_ref[pl.ds(bt_ref[pp, b], 1), :, :]
            parA_ref[b * _A:(b + 1) * _A, :] = chunk[0]
        ccx = crdc_ref[0, :, 0:1]
        ccy = crdc_ref[0, :, 1:2]
        ccz = crdc_ref[0, :, 2:3]
        ncx = nbc_ref[0, :, 0:1]
        ncy = nbc_ref[0, :, 1:2]
        ncz = nbc_ref[0, :, 2:3]
        q1x, q1y, q1z, q2x, q2y, q2z = _water_dirs(ccx, ccy, ccz, ncx, ncy, ncz)
        ctc = jnp.cos(torsc_ref[...])
        stc = jnp.sin(torsc_ref[...])
        for w in range(_W):
            cw = ctc[0:1, w:w + 1]
            sw = stc[0:1, w:w + 1]
            wc3_ref[w * _N:(w + 1) * _N, 0:1] = ccx + wd * (q1x * cw + q2x * sw)
            wc3_ref[w * _N:(w + 1) * _N, 1:2] = ccy + wd * (q1y * cw + q2y * sw)
            wc3_ref[w * _N:(w + 1) * _N, 2:3] = ccz + wd * (q1z * cw + q2z * sw)
        rcx = crdr_ref[0, 0:1, :]
        rcy = crdr_ref[0, 1:2, :]
        rcz = crdr_ref[0, 2:3, :]
        rnx = nbr_ref[0, 0:1, :]
        rny = nbr_ref[0, 1:2, :]
        rnz = nbr_ref[0, 2:3, :]
        r1x, r1y, r1z, r2x, r2y, r2z = _water_dirs(rcx, rcy, rcz, rnx, rny, rnz)
        t0 = torsr_ref[0:1, 0:1]
        ct0 = jnp.cos(t0)
        st0 = jnp.sin(t0)
        w0r_ref[0:1, :] = rcx + wd * (r1x * ct0 + r2x * st0)
        w0r_ref[1:2, :] = rcy + wd * (r1y * ct0 + r2y * st0)
        w0r_ref[2:3, :] = rcz + wd * (r1z * ct0 + r2z * st0)

    intra = jnp.concatenate(
        [pathN_ref[pl.ds(bt_ref[pp, t * _TB + lb], 1), :, :][0]
         for lb in range(_TB)], axis=0)
    inter = jnp.concatenate(
        [jnp.broadcast_to(msbN_ref[0, pl.ds(t * _TB + lb, 1), :], (_A, _N))
         for lb in range(_TB)], axis=0)
    ri = r0 + jax.lax.broadcasted_iota(jnp.int32, (_TI, _N), 0)
    cj = jax.lax.broadcasted_iota(jnp.int32, (_TI, _N), 1)
    same_blk = (ri // _A) == (cj // _A)
    sep = jnp.where(same_blk, intra, inter)
    hvy_j = parN_ref[4:5, :]
    hvy_i = parA_ref[pl.ds(r0, _TI), 4:5]
    pm = jnp.where((sep >= 4.0) & (ri != cj), 1.0, 0.0) * (hvy_i * hvy_j)

    cr = crdr_ref[0]
    ci = crdc_ref[0, pl.ds(r0, _TI), :]
    sq_j = (cr[0:1, :] * cr[0:1, :] + cr[1:2, :] * cr[1:2, :]
            + cr[2:3, :] * cr[2:3, :])
    sq_i = (ci[:, 0:1] * ci[:, 0:1] + ci[:, 1:2] * ci[:, 1:2]
            + ci[:, 2:3] * ci[:, 2:3])
    xx = jnp.dot(ci, cr, preferred_element_type=f32)
    d = jnp.sqrt(jnp.maximum(sq_i + sq_j - 2.0 * xx, 1e-6))
    dg_j = parN_ref[0:1, :]
    rad_j = parN_ref[2:3, :]
    pol_j = parN_ref[3:4, :]
    dg_i = parA_ref[pl.ds(r0, _TI), 0:1]
    ilam_i = parA_ref[pl.ds(r0, _TI), 1:2]
    rad_i = parA_ref[pl.ds(r0, _TI), 2:3]
    pol_i = parA_ref[pl.ds(r0, _TI), 3:4]
    xq = (d - (rad_i + rad_j)) * ilam_i
    lk_iso_p = dg_i * jnp.exp(-(xq * xq)) * pm * pol_i
    s_iso = jnp.sum(lk_iso_p)

    rr = ramp_ref[0, 0]
    inv_r2 = 1.0 / (rr * rr)
    occ = jnp.zeros((_TI, _N), f32)
    for w in range(_W):
        wi = wc3_ref[pl.ds(w * _N + r0, _TI), :]
        wsq_i = (wi[:, 0:1] * wi[:, 0:1] + wi[:, 1:2] * wi[:, 1:2]
                 + wi[:, 2:3] * wi[:, 2:3])
        wxx = jnp.dot(wi, cr, preferred_element_type=f32)
        d2w = jnp.maximum(wsq_i + sq_j - 2.0 * wxx, 1e-6)
        occ = occ + jnp.exp(-d2w * inv_r2)
    s_ball = jnp.sum(lk_iso_p * jnp.minimum(occ, 1.0))

    w0r = w0r_ref[...]
    w0i = wc3_ref[pl.ds(r0, _TI), :]
    w0sq_j = (w0r[0:1, :] * w0r[0:1, :] + w0r[1:2, :] * w0r[1:2, :]
              + w0r[2:3, :] * w0r[2:3, :])
    w0sq_i = (w0i[:, 0:1] * w0i[:, 0:1] + w0i[:, 1:2] * w0i[:, 1:2]
              + w0i[:, 2:3] * w0i[:, 2:3])
    bxx = jnp.dot(w0i, w0r, preferred_element_type=f32)
    d2b = jnp.maximum(w0sq_i + w0sq_j - 2.0 * bxx, 1e-6)
    bp = jnp.exp(-d2b * inv_r2) * pm * pol_i * pol_j
    s_br = jnp.sum(0.5 * (dg_i + dg_j) * bp)
    s_un = jnp.sum(bp)

    i4 = jax.lax.broadcasted_iota(jnp.int32, (1, 4), 1)
    vec = (jnp.where(i4 == 0, s_iso, 0.0) + jnp.where(i4 == 1, s_ball, 0.0)
           + jnp.where(i4 == 2, s_br, 0.0) + jnp.where(i4 == 3, s_un, 0.0))

    @pl.when(t == 0)
    def _():
        out_ref[0] = vec

    @pl.when(t > 0)
    def _():
        out_ref[0] = out_ref[0] + vec


def kernel(pose_coords, pose_stack_block_coord_offset, pose_stack_block_type,
           pose_stack_min_bond_separation, bt_path_distance, bt_atom_is_hydrogen,
           bt_tile_lk_ball_params, lk_ball_global_params, water_gen_global_params,
           sp2_water_tors, sp3_water_tors, ring_water_tors):
    f32 = jnp.float32
    crd_c = pose_coords.astype(f32)
    nb_c = jnp.roll(crd_c, 1, axis=1)
    crd_r = jnp.transpose(crd_c, (0, 2, 1))
    nb_r = jnp.transpose(nb_c, (0, 2, 1))
    prm = bt_tile_lk_ball_params[:, 0].astype(f32)
    is_h0 = (bt_atom_is_hydrogen == 0)
    tblA = jnp.stack([
        prm[..., 0] * 5.0,
        1.0 / (prm[..., 1] * 2.0 + 1.0),
        prm[..., 3] + 1.0,
        (((prm[..., 4] > 0.5) | (prm[..., 7] > 0.5)) & is_h0).astype(f32),
        is_h0.astype(f32),
    ], axis=-1)
    tblN = jnp.tile(jnp.transpose(tblA, (2, 0, 1)), (1, 1, _B))
    pathN = jnp.tile(bt_path_distance.astype(f32), (1, 1, _B))
    msbN = jnp.repeat(pose_stack_min_bond_separation.astype(f32),
                      _A, axis=2)
    torsv = jnp.concatenate([sp2_water_tors, sp3_water_tors]).astype(f32)
    torsc = torsv.reshape(1, _W)
    torsr = torsv.reshape(_W, 1)
    bt_s = pose_stack_block_type.astype(jnp.int32)
    wdist = water_gen_global_params[:1, :1].astype(f32)
    ramp = lk_ball_global_params[:1, :1].astype(f32)

    out = pl.pallas_call(
        _body,
        grid=(_P, _T),
        in_specs=[
            pl.BlockSpec((1, 3, _N), lambda p, t: (p, 0, 0)),
            pl.BlockSpec((1, 3, _N), lambda p, t: (p, 0, 0)),
            pl.BlockSpec((1, _N, 3), lambda p, t: (p, 0, 0)),
            pl.BlockSpec((1, _N, 3), lambda p, t: (p, 0, 0)),
            pl.BlockSpec((5, _NBT, _N), lambda p, t: (0, 0, 0)),
            pl.BlockSpec((_NBT, _A, 5), lambda p, t: (0, 0, 0)),
            pl.BlockSpec((_NBT, _A, _N), lambda p, t: (0, 0, 0)),
            pl.BlockSpec((1, _B, _N), lambda p, t: (p, 0, 0)),
            pl.BlockSpec((1, _W), lambda p, t: (0, 0)),
            pl.BlockSpec((_W, 1), lambda p, t: (0, 0)),
            pl.BlockSpec(memory_space=pltpu.SMEM),
            pl.BlockSpec(memory_space=pltpu.SMEM),
            pl.BlockSpec(memory_space=pltpu.SMEM),
        ],
        out_specs=pl.BlockSpec((1, 1, 4), lambda p, t: (p, 0, 0)),
        out_shape=jax.ShapeDtypeStruct((_P, 1, 4), f32),
        scratch_shapes=[
            pltpu.VMEM((_W * _N, 3), f32),
            pltpu.VMEM((3, _N), f32),
            pltpu.VMEM((5, _N), f32),
            pltpu.VMEM((_N, 5), f32),
        ],
    )(crd_r, nb_r, crd_c, nb_c, tblN, tblA, pathN, msbN, torsc, torsr,
      bt_s, wdist, ramp)
    return out.reshape(_P, 4).T


# speedup vs baseline: 1170.5244x; 1.2955x over previous
import jax
import jax.numpy as jnp
from jax.experimental import pallas as pl
from jax.experimental.pallas import tpu as pltpu

_P, _B, _A, _NBT, _W = 4, 32, 32, 24, 4
_N = _B * _A
_TI = 256
_TB = _TI // _A
_T = _N // _TI
_RX, _RY, _RZ = 0.057, 0.31, 0.95

_DN = (((0,), (0,)), ((), ()))


def _body(crdr_ref, nbr_ref, tblN_ref, tblA_ref,
          pathN_ref, msbN_ref, torsc_ref,
          bt_ref, wdist_ref, ramp_ref, out_ref,
          wrc_ref, aug_ref, bocc_ref, bw0_ref, parN_ref, parA_ref):
    f32 = jnp.float32
    pp = pl.program_id(0)
    t = pl.program_id(1)
    r0 = t * _TI
    wd = wdist_ref[0, 0]

    @pl.when(t == 0)
    def _():
        jva = jax.lax.broadcasted_iota(jnp.int32, (1, _N), 1) // _A
        btN = jnp.zeros((1, _N), jnp.int32)
        for b in range(_B):
            btN = jnp.where(jva == b, bt_ref[pp, b], btN)
        for k in range(5):
            acc = jnp.zeros((1, _N), f32)
            for tt in range(_NBT):
                acc = jnp.where(btN == tt, tblN_ref[k, tt:tt + 1, :], acc)
            parN_ref[k:k + 1, :] = acc
        for b in range(_B):
            chunk = tblA_ref[pl.ds(bt_ref[pp, b], 1), :, :]
            parA_ref[b * _A:(b + 1) * _A, :] = chunk[0]
        cx = crdr_ref[0, 0:1, :]
        cy = crdr_ref[0, 1:2, :]
        cz = crdr_ref[0, 2:3, :]
        aug_ref[0:3, :] = crdr_ref[0]
        aug_ref[3:4, :] = cx * cx + cy * cy + cz * cz
        bocc_ref[0:3, :] = crdr_ref[0] * -2.0
        bocc_ref[3:4, :] = jnp.ones((1, _N), f32)
        nx = nbr_ref[0, 0:1, :]
        ny = nbr_ref[0, 1:2, :]
        nz = nbr_ref[0, 2:3, :]
        vx, vy, vz = cx - nx, cy - ny, cz - nz
        s = 1.0 / (jnp.sqrt(vx * vx + vy * vy + vz * vz) + 1e-6)
        vx, vy, vz = vx * s, vy * s, vz * s
        p1x = vy * _RZ - vz * _RY
        p1y = vz * _RX - vx * _RZ
        p1z = vx * _RY - vy * _RX
        s1 = 1.0 / (jnp.sqrt(p1x * p1x + p1y * p1y + p1z * p1z) + 1e-6)
        p1x, p1y, p1z = p1x * s1, p1y * s1, p1z * s1
        p2x = vy * p1z - vz * p1y
        p2y = vz * p1x - vx * p1z
        p2z = vx * p1y - vy * p1x
        ct = jnp.cos(torsc_ref[...])
        st = jnp.sin(torsc_ref[...])
        for w in range(_W):
            cw = ct[0:1, w:w + 1]
            sw = st[0:1, w:w + 1]
            wx = cx + wd * (p1x * cw + p2x * sw)
            wy = cy + wd * (p1y * cw + p2y * sw)
            wz = cz + wd * (p1z * cw + p2z * sw)
            wrc_ref[4 * w + 0:4 * w + 1, :] = wx
            wrc_ref[4 * w + 1:4 * w + 2, :] = wy
            wrc_ref[4 * w + 2:4 * w + 3, :] = wz
            wrc_ref[4 * w + 3:4 * w + 4, :] = wx * wx + wy * wy + wz * wz
        bw0_ref[0:3, :] = wrc_ref[0:3, :] * -2.0
        bw0_ref[3:4, :] = jnp.ones((1, _N), f32)

    intra = jnp.concatenate(
        [pathN_ref[pl.ds(bt_ref[pp, t * _TB + lb], 1), :, :][0]
         for lb in range(_TB)], axis=0)
    inter = jnp.concatenate(
        [jnp.broadcast_to(msbN_ref[0, pl.ds(t * _TB + lb, 1), :], (_A, _N))
         for lb in range(_TB)], axis=0)
    ri = r0 + jax.lax.broadcasted_iota(jnp.int32, (_TI, _N), 0)
    cj = jax.lax.broadcasted_iota(jnp.int32, (_TI, _N), 1)
    same_blk = (ri // _A) == (cj // _A)
    sep = jnp.where(same_blk, intra, inter)
    hvy_j = parN_ref[4:5, :]
    hvy_i = parA_ref[pl.ds(r0, _TI), 4:5]
    pm = jnp.where((sep >= 4.0) & (ri != cj), 1.0, 0.0) * (hvy_i * hvy_j)

    f32 = jnp.float32
    a_at = aug_ref[:, pl.ds(r0, _TI)]
    sq_j = aug_ref[3:4, :]
    d2 = jax.lax.dot_general(a_at, bocc_ref[...], _DN,
                             preferred_element_type=f32) + sq_j
    d = jnp.sqrt(jnp.maximum(d2, 1e-6))
    dg_j = parN_ref[0:1, :]
    rad_j = parN_ref[2:3, :]
    pol_j = parN_ref[3:4, :]
    dg_i = parA_ref[pl.ds(r0, _TI), 0:1]
    ilam_i = parA_ref[pl.ds(r0, _TI), 1:2]
    rad_i = parA_ref[pl.ds(r0, _TI), 2:3]
    pol_i = parA_ref[pl.ds(r0, _TI), 3:4]
    pmp = pm * pol_i
    xq = (d - (rad_i + rad_j)) * ilam_i
    lk_iso_p = jnp.exp(-(xq * xq)) * pmp * dg_i
    s_iso = jnp.sum(lk_iso_p)

    rr = ramp_ref[0, 0]
    inv_r2 = 1.0 / (rr * rr)
    occ = jnp.zeros((_TI, _N), f32)
    for w in range(_W):
        a_w = wrc_ref[4 * w:4 * w + 4, pl.ds(r0, _TI)]
        d2w = jnp.maximum(jax.lax.dot_general(a_w, bocc_ref[...], _DN,
                                              preferred_element_type=f32)
                          + sq_j, 1e-6)
        occ = occ + jnp.exp(-d2w * inv_r2)
    s_ball = jnp.sum(lk_iso_p * jnp.minimum(occ, 1.0))

    a_w0 = wrc_ref[0:4, pl.ds(r0, _TI)]
    w0sq_j = wrc_ref[3:4, :]
    d2b = jnp.maximum(jax.lax.dot_general(a_w0, bw0_ref[...], _DN,
                                          preferred_element_type=f32)
                      + w0sq_j, 1e-6)
    bp = jnp.exp(-d2b * inv_r2) * pmp * pol_j
    s_br = jnp.sum(0.5 * (dg_i + dg_j) * bp)
    s_un = jnp.sum(bp)

    i4 = jax.lax.broadcasted_iota(jnp.int32, (1, 4), 1)
    vec = (jnp.where(i4 == 0, s_iso, 0.0) + jnp.where(i4 == 1, s_ball, 0.0)
           + jnp.where(i4 == 2, s_br, 0.0) + jnp.where(i4 == 3, s_un, 0.0))

    @pl.when(t == 0)
    def _():
        out_ref[0] = vec

    @pl.when(t > 0)
    def _():
        out_ref[0] = out_ref[0] + vec


def kernel(pose_coords, pose_stack_block_coord_offset, pose_stack_block_type,
           pose_stack_min_bond_separation, bt_path_distance, bt_atom_is_hydrogen,
           bt_tile_lk_ball_params, lk_ball_global_params, water_gen_global_params,
           sp2_water_tors, sp3_water_tors, ring_water_tors):
    f32 = jnp.float32
    crd_c = pose_coords.astype(f32)
    nb_c = jnp.roll(crd_c, 1, axis=1)
    crd_r = jnp.transpose(crd_c, (0, 2, 1))
    nb_r = jnp.transpose(nb_c, (0, 2, 1))
    prm = bt_tile_lk_ball_params[:, 0].astype(f32)
    is_h0 = (bt_atom_is_hydrogen == 0)
    tblA = jnp.stack([
        prm[..., 0] * 5.0,
        1.0 / (prm[..., 1] * 2.0 + 1.0),
        prm[..., 3] + 1.0,
        (((prm[..., 4] > 0.5) | (prm[..., 7] > 0.5)) & is_h0).astype(f32),
        is_h0.astype(f32),
    ], axis=-1)
    tblN = jnp.tile(jnp.transpose(tblA, (2, 0, 1)), (1, 1, _B))
    pathN = jnp.tile(bt_path_distance.astype(f32), (1, 1, _B))
    msbN = jnp.repeat(pose_stack_min_bond_separation.astype(f32),
                      _A, axis=2)
    torsv = jnp.concatenate([sp2_water_tors, sp3_water_tors]).astype(f32)
    torsc = torsv.reshape(1, _W)
    bt_s = pose_stack_block_type.astype(jnp.int32)
    wdist = water_gen_global_params[:1, :1].astype(f32)
    ramp = lk_ball_global_params[:1, :1].astype(f32)

    out = pl.pallas_call(
        _body,
        grid=(_P, _T),
        in_specs=[
            pl.BlockSpec((1, 3, _N), lambda p, t: (p, 0, 0)),
            pl.BlockSpec((1, 3, _N), lambda p, t: (p, 0, 0)),
            pl.BlockSpec((5, _NBT, _N), lambda p, t: (0, 0, 0)),
            pl.BlockSpec((_NBT, _A, 5), lambda p, t: (0, 0, 0)),
            pl.BlockSpec((_NBT, _A, _N), lambda p, t: (0, 0, 0)),
            pl.BlockSpec((1, _B, _N), lambda p, t: (p, 0, 0)),
            pl.BlockSpec((1, _W), lambda p, t: (0, 0)),
            pl.BlockSpec(memory_space=pltpu.SMEM),
            pl.BlockSpec(memory_space=pltpu.SMEM),
            pl.BlockSpec(memory_space=pltpu.SMEM),
        ],
        out_specs=pl.BlockSpec((1, 1, 4), lambda p, t: (p, 0, 0)),
        out_shape=jax.ShapeDtypeStruct((_P, 1, 4), f32),
        scratch_shapes=[
            pltpu.VMEM((4 * _W, _N), f32),
            pltpu.VMEM((4, _N), f32),
            pltpu.VMEM((4, _N), f32),
            pltpu.VMEM((4, _N), f32),
            pltpu.VMEM((5, _N), f32),
            pltpu.VMEM((_N, 5), f32),
        ],
    )(crd_r, nb_r, tblN, tblA, pathN, msbN, torsc,
      bt_s, wdist, ramp)
    return out.reshape(_P, 4).T


# speedup vs baseline: 1170.7370x; 1.0002x over previous
import jax
import jax.numpy as jnp
from jax.experimental import pallas as pl
from jax.experimental.pallas import tpu as pltpu

_P, _B, _A, _NBT, _W = 4, 32, 32, 24, 4
_N = _B * _A
_TI = 256
_TB = _TI // _A
_T = _N // _TI
_RX, _RY, _RZ = 0.057, 0.31, 0.95

_DN = (((0,), (0,)), ((), ()))


def _body(crdr_ref, nbr_ref, tblN_ref, tblA_ref,
          pathN_ref, msbN_ref, torsc_ref,
          bt_ref, wdist_ref, ramp_ref, out_ref,
          wrc_ref, aug_ref, bocc_ref, bw0_ref, parN_ref, parA_ref):
    f32 = jnp.float32
    pp = pl.program_id(0)
    t = pl.program_id(1)
    r0 = t * _TI
    wd = wdist_ref[0, 0]

    @pl.when(t == 0)
    def _():
        jva = jax.lax.broadcasted_iota(jnp.int32, (1, _N), 1) // _A
        btN = jnp.zeros((1, _N), jnp.int32)
        for b in range(_B):
            btN = jnp.where(jva == b, bt_ref[pp, b], btN)
        for k in range(5):
            acc = jnp.zeros((1, _N), f32)
            for tt in range(_NBT):
                acc = jnp.where(btN == tt, tblN_ref[k, tt:tt + 1, :], acc)
            parN_ref[k:k + 1, :] = acc
        for b in range(_B):
            chunk = tblA_ref[pl.ds(bt_ref[pp, b], 1), :, :]
            parA_ref[b * _A:(b + 1) * _A, :] = chunk[0]
        cx = crdr_ref[0, 0:1, :]
        cy = crdr_ref[0, 1:2, :]
        cz = crdr_ref[0, 2:3, :]
        aug_ref[0:3, :] = crdr_ref[0]
        aug_ref[3:4, :] = cx * cx + cy * cy + cz * cz
        bocc_ref[0:3, :] = crdr_ref[0] * -2.0
        bocc_ref[3:4, :] = jnp.ones((1, _N), f32)
        nx = nbr_ref[0, 0:1, :]
        ny = nbr_ref[0, 1:2, :]
        nz = nbr_ref[0, 2:3, :]
        vx, vy, vz = cx - nx, cy - ny, cz - nz
        s = 1.0 / (jnp.sqrt(vx * vx + vy * vy + vz * vz) + 1e-6)
        vx, vy, vz = vx * s, vy * s, vz * s
        p1x = vy * _RZ - vz * _RY
        p1y = vz * _RX - vx * _RZ
        p1z = vx * _RY - vy * _RX
        s1 = 1.0 / (jnp.sqrt(p1x * p1x + p1y * p1y + p1z * p1z) + 1e-6)
        p1x, p1y, p1z = p1x * s1, p1y * s1, p1z * s1
        p2x = vy * p1z - vz * p1y
        p2y = vz * p1x - vx * p1z
        p2z = vx * p1y - vy * p1x
        ct = jnp.cos(torsc_ref[...])
        st = jnp.sin(torsc_ref[...])
        for w in range(_W):
            cw = ct[0:1, w:w + 1]
            sw = st[0:1, w:w + 1]
            wx = cx + wd * (p1x * cw + p2x * sw)
            wy = cy + wd * (p1y * cw + p2y * sw)
            wz = cz + wd * (p1z * cw + p2z * sw)
            wrc_ref[4 * w + 0:4 * w + 1, :] = wx
            wrc_ref[4 * w + 1:4 * w + 2, :] = wy
            wrc_ref[4 * w + 2:4 * w + 3, :] = wz
            wrc_ref[4 * w + 3:4 * w + 4, :] = wx * wx + wy * wy + wz * wz
        bw0_ref[0:3, :] = wrc_ref[0:3, :] * -2.0
        bw0_ref[3:4, :] = jnp.ones((1, _N), f32)

    intra = jnp.concatenate(
        [pathN_ref[pl.ds(bt_ref[pp, t * _TB + lb], 1), :, :][0]
         for lb in range(_TB)], axis=0)
    inter = jnp.concatenate(
        [jnp.broadcast_to(msbN_ref[0, pl.ds(t * _TB + lb, 1), :], (_A, _N))
         for lb in range(_TB)], axis=0)
    ri = r0 + jax.lax.broadcasted_iota(jnp.int32, (_TI, _N), 0)
    cj = jax.lax.broadcasted_iota(jnp.int32, (_TI, _N), 1)
    same_blk = (ri // _A) == (cj // _A)
    sep = jnp.where(same_blk, intra, inter)
    hvy_j = parN_ref[4:5, :]
    hvy_i = parA_ref[pl.ds(r0, _TI), 4:5]
    pm = jnp.where((sep >= 4.0) & (ri != cj), 1.0, 0.0) * (hvy_i * hvy_j)

    f32 = jnp.float32
    a_at = aug_ref[:, pl.ds(r0, _TI)]
    sq_j = aug_ref[3:4, :]
    d2 = jax.lax.dot_general(a_at, bocc_ref[...], _DN,
                             preferred_element_type=f32) + sq_j
    d = jnp.sqrt(jnp.maximum(d2, 1e-6))
    dg_j = parN_ref[0:1, :]
    rad_j = parN_ref[2:3, :]
    pol_j = parN_ref[3:4, :]
    dg_i = parA_ref[pl.ds(r0, _TI), 0:1]
    ilam_i = parA_ref[pl.ds(r0, _TI), 1:2]
    rad_i = parA_ref[pl.ds(r0, _TI), 2:3]
    pol_i = parA_ref[pl.ds(r0, _TI), 3:4]
    pmp = pm * pol_i
    xq = (d - (rad_i + rad_j)) * ilam_i
    lk_iso_p = jnp.exp(-(xq * xq)) * pmp * dg_i
    s_iso = jnp.sum(lk_iso_p)

    rr = ramp_ref[0, 0]
    inv_r2 = 1.0 / (rr * rr)
    occ = jnp.zeros((_TI, _N), f32)
    for w in range(_W):
        a_w = wrc_ref[4 * w:4 * w + 4, pl.ds(r0, _TI)]
        d2w = jnp.maximum(jax.lax.dot_general(a_w, bocc_ref[...], _DN,
                                              preferred_element_type=f32)
                          + sq_j, 1e-6)
        occ = occ + jnp.exp(-d2w * inv_r2)
    s_ball = jnp.sum(lk_iso_p * jnp.minimum(occ, 1.0))

    a_w0 = wrc_ref[0:4, pl.ds(r0, _TI)]
    w0sq_j = wrc_ref[3:4, :]
    d2b = jnp.maximum(jax.lax.dot_general(a_w0, bw0_ref[...], _DN,
                                          preferred_element_type=f32)
                      + w0sq_j, 1e-6)
    bp = jnp.exp(-d2b * inv_r2) * pmp * pol_j
    s_br = jnp.sum(0.5 * (dg_i + dg_j) * bp)
    s_un = jnp.sum(bp)

    i4 = jax.lax.broadcasted_iota(jnp.int32, (1, 4), 1)
    vec = (jnp.where(i4 == 0, s_iso, 0.0) + jnp.where(i4 == 1, s_ball, 0.0)
           + jnp.where(i4 == 2, s_br, 0.0) + jnp.where(i4 == 3, s_un, 0.0))

    @pl.when(t == 0)
    def _():
        out_ref[0] = vec

    @pl.when(t > 0)
    def _():
        out_ref[0] = out_ref[0] + vec


def kernel(pose_coords, pose_stack_block_coord_offset, pose_stack_block_type,
           pose_stack_min_bond_separation, bt_path_distance, bt_atom_is_hydrogen,
           bt_tile_lk_ball_params, lk_ball_global_params, water_gen_global_params,
           sp2_water_tors, sp3_water_tors, ring_water_tors):
    f32 = jnp.float32
    crd_c = pose_coords.astype(f32)
    nb_c = jnp.roll(crd_c, 1, axis=1)
    crd_r = jnp.transpose(crd_c, (0, 2, 1))
    nb_r = jnp.transpose(nb_c, (0, 2, 1))
    prm = bt_tile_lk_ball_params[:, 0].astype(f32)
    is_h0 = (bt_atom_is_hydrogen == 0)
    tblA = jnp.stack([
        prm[..., 0] * 5.0,
        1.0 / (prm[..., 1] * 2.0 + 1.0),
        prm[..., 3] + 1.0,
        (((prm[..., 4] > 0.5) | (prm[..., 7] > 0.5)) & is_h0).astype(f32),
        is_h0.astype(f32),
    ], axis=-1)
    tblN = jnp.tile(jnp.transpose(tblA, (2, 0, 1)), (1, 1, _B))
    pathN = jnp.tile(bt_path_distance.astype(f32), (1, 1, _B))
    msbN = jnp.repeat(pose_stack_min_bond_separation.astype(f32),
                      _A, axis=2)
    torsv = jnp.concatenate([sp2_water_tors, sp3_water_tors]).astype(f32)
    torsc = torsv.reshape(1, _W)
    bt_s = pose_stack_block_type.astype(jnp.int32)
    wdist = water_gen_global_params[:1, :1].astype(f32)
    ramp = lk_ball_global_params[:1, :1].astype(f32)

    out = pl.pallas_call(
        _body,
        grid=(_P, _T),
        compiler_params=pltpu.CompilerParams(
            dimension_semantics=("parallel", "arbitrary")),
        in_specs=[
            pl.BlockSpec((1, 3, _N), lambda p, t: (p, 0, 0)),
            pl.BlockSpec((1, 3, _N), lambda p, t: (p, 0, 0)),
            pl.BlockSpec((5, _NBT, _N), lambda p, t: (0, 0, 0)),
            pl.BlockSpec((_NBT, _A, 5), lambda p, t: (0, 0, 0)),
            pl.BlockSpec((_NBT, _A, _N), lambda p, t: (0, 0, 0)),
            pl.BlockSpec((1, _B, _N), lambda p, t: (p, 0, 0)),
            pl.BlockSpec((1, _W), lambda p, t: (0, 0)),
            pl.BlockSpec(memory_space=pltpu.SMEM),
            pl.BlockSpec(memory_space=pltpu.SMEM),
            pl.BlockSpec(memory_space=pltpu.SMEM),
        ],
        out_specs=pl.BlockSpec((1, 1, 4), lambda p, t: (p, 0, 0)),
        out_shape=jax.ShapeDtypeStruct((_P, 1, 4), f32),
        scratch_shapes=[
            pltpu.VMEM((4 * _W, _N), f32),
            pltpu.VMEM((4, _N), f32),
            pltpu.VMEM((4, _N), f32),
            pltpu.VMEM((4, _N), f32),
            pltpu.VMEM((5, _N), f32),
            pltpu.VMEM((_N, 5), f32),
        ],
    )(crd_r, nb_r, tblN, tblA, pathN, msbN, torsc,
      bt_s, wdist, ramp)
    return out.reshape(_P, 4).T


# speedup vs baseline: 1377.6629x; 1.1767x over previous
import jax
import jax.numpy as jnp
from jax.experimental import pallas as pl
from jax.experimental.pallas import tpu as pltpu

_P, _B, _A, _NBT, _W = 4, 32, 32, 24, 4
_N = _B * _A
_TI = 256
_TB = _TI // _A
_T = _N // _TI
_RX, _RY, _RZ = 0.057, 0.31, 0.95
_LOG2E = 1.4426950408889634

_DN = (((0,), (0,)), ((), ()))


def _body(crdr_ref, nbr_ref, tblN_ref, tblA_ref,
          pathN_ref, msbN_ref, torsc_ref,
          bt_ref, wdist_ref, ramp_ref, out_ref,
          wrc_ref, aug_ref, bocc_ref, bw0_ref, parN_ref, parA_ref):
    f32 = jnp.float32
    pp = pl.program_id(0)
    t = pl.program_id(1)
    r0 = t * _TI
    wd = wdist_ref[0, 0]

    @pl.when(t == 0)
    def _():
        jva = jax.lax.broadcasted_iota(jnp.int32, (1, _N), 1) // _A
        btN = jnp.zeros((1, _N), jnp.int32)
        for b in range(_B):
            btN = jnp.where(jva == b, bt_ref[pp, b], btN)
        for k in range(5):
            acc = jnp.zeros((1, _N), f32)
            for tt in range(_NBT):
                acc = jnp.where(btN == tt, tblN_ref[k, tt:tt + 1, :], acc)
            parN_ref[k:k + 1, :] = acc
        for b in range(_B):
            chunk = tblA_ref[pl.ds(bt_ref[pp, b], 1), :, :]
            parA_ref[b * _A:(b + 1) * _A, :] = chunk[0]
        one = jnp.ones((1, _N), f32)
        cx = crdr_ref[0, 0:1, :]
        cy = crdr_ref[0, 1:2, :]
        cz = crdr_ref[0, 2:3, :]
        csq = cx * cx + cy * cy + cz * cz
        aug_ref[0:3, :] = crdr_ref[0]
        aug_ref[3:4, :] = csq
        aug_ref[4:5, :] = one
        bocc_ref[0:3, :] = crdr_ref[0] * -2.0
        bocc_ref[3:4, :] = one
        bocc_ref[4:5, :] = csq
        nx = nbr_ref[0, 0:1, :]
        ny = nbr_ref[0, 1:2, :]
        nz = nbr_ref[0, 2:3, :]
        vx, vy, vz = cx - nx, cy - ny, cz - nz
        s = 1.0 / (jnp.sqrt(vx * vx + vy * vy + vz * vz) + 1e-6)
        vx, vy, vz = vx * s, vy * s, vz * s
        p1x = vy * _RZ - vz * _RY
        p1y = vz * _RX - vx * _RZ
        p1z = vx * _RY - vy * _RX
        s1 = 1.0 / (jnp.sqrt(p1x * p1x + p1y * p1y + p1z * p1z) + 1e-6)
        p1x, p1y, p1z = p1x * s1, p1y * s1, p1z * s1
        p2x = vy * p1z - vz * p1y
        p2y = vz * p1x - vx * p1z
        p2z = vx * p1y - vy * p1x
        ct = jnp.cos(torsc_ref[...])
        st = jnp.sin(torsc_ref[...])
        for w in range(_W):
            cw = ct[0:1, w:w + 1]
            sw = st[0:1, w:w + 1]
            wx = cx + wd * (p1x * cw + p2x * sw)
            wy = cy + wd * (p1y * cw + p2y * sw)
            wz = cz + wd * (p1z * cw + p2z * sw)
            wrc_ref[5 * w + 0:5 * w + 1, :] = wx
            wrc_ref[5 * w + 1:5 * w + 2, :] = wy
            wrc_ref[5 * w + 2:5 * w + 3, :] = wz
            wrc_ref[5 * w + 3:5 * w + 4, :] = wx * wx + wy * wy + wz * wz
            wrc_ref[5 * w + 4:5 * w + 5, :] = one
        bw0_ref[0:3, :] = wrc_ref[0:3, :] * -2.0
        bw0_ref[3:4, :] = one
        bw0_ref[4:5, :] = wrc_ref[3:4, :]

    intra = jnp.concatenate(
        [pathN_ref[pl.ds(bt_ref[pp, t * _TB + lb], 1), :, :][0]
         for lb in range(_TB)], axis=0)
    inter = jnp.concatenate(
        [jnp.broadcast_to(msbN_ref[0, pl.ds(t * _TB + lb, 1), :], (_A, _N))
         for lb in range(_TB)], axis=0)
    ri = r0 + jax.lax.broadcasted_iota(jnp.int32, (_TI, _N), 0)
    cj = jax.lax.broadcasted_iota(jnp.int32, (_TI, _N), 1)
    same_blk = (ri // _A) == (cj // _A)
    sep = jnp.where(same_blk, intra, inter)
    hvy_j = parN_ref[4:5, :]
    pol_i = parA_ref[pl.ds(r0, _TI), 3:4]
    pmp = jnp.where((sep >= 4.0) & (ri != cj), 1.0, 0.0) * (pol_i * hvy_j)

    a_at = aug_ref[:, pl.ds(r0, _TI)]
    d2 = jax.lax.dot_general(a_at, bocc_ref[...], _DN,
                             preferred_element_type=f32)
    d = jnp.sqrt(jnp.maximum(d2, 1e-6))
    dg_j = parN_ref[0:1, :]
    rad_j = parN_ref[2:3, :]
    pol_j = parN_ref[3:4, :]
    dg_i = parA_ref[pl.ds(r0, _TI), 0:1]
    nl2_i = parA_ref[pl.ds(r0, _TI), 1:2]
    rad_i = parA_ref[pl.ds(r0, _TI), 2:3]
    dm = d - (rad_i + rad_j)
    lk_iso_p = jnp.exp2(dm * dm * nl2_i) * pmp * dg_i
    s_iso = jnp.sum(lk_iso_p)

    rr = ramp_ref[0, 0]
    c2 = -_LOG2E / (rr * rr)
    occ = jnp.zeros((_TI, _N), f32)
    for w in range(_W):
        a_w = wrc_ref[5 * w:5 * w + 5, pl.ds(r0, _TI)]
        d2w = jax.lax.dot_general(a_w, bocc_ref[...], _DN,
                                  preferred_element_type=f32)
        occ = occ + jnp.exp2(d2w * c2)
    s_ball = jnp.sum(lk_iso_p * jnp.minimum(occ, 1.0))

    a_w0 = wrc_ref[0:5, pl.ds(r0, _TI)]
    d2b = jax.lax.dot_general(a_w0, bw0_ref[...], _DN,
                              preferred_element_type=f32)
    bp = jnp.exp2(d2b * c2) * pmp * pol_j
    s_br = jnp.sum((0.5 * dg_i + 0.5 * dg_j) * bp)
    s_un = jnp.sum(bp)

    i4 = jax.lax.broadcasted_iota(jnp.int32, (1, 4), 1)
    vec = (jnp.where(i4 == 0, s_iso, 0.0) + jnp.where(i4 == 1, s_ball, 0.0)
           + jnp.where(i4 == 2, s_br, 0.0) + jnp.where(i4 == 3, s_un, 0.0))

    @pl.when(t == 0)
    def _():
        out_ref[0] = vec

    @pl.when(t > 0)
    def _():
        out_ref[0] = out_ref[0] + vec


def kernel(pose_coords, pose_stack_block_coord_offset, pose_stack_block_type,
           pose_stack_min_bond_separation, bt_path_distance, bt_atom_is_hydrogen,
           bt_tile_lk_ball_params, lk_ball_global_params, water_gen_global_params,
           sp2_water_tors, sp3_water_tors, ring_water_tors):
    f32 = jnp.float32
    crd_c = pose_coords.astype(f32)
    nb_c = jnp.roll(crd_c, 1, axis=1)
    crd_r = jnp.transpose(crd_c, (0, 2, 1))
    nb_r = jnp.transpose(nb_c, (0, 2, 1))
    prm = bt_tile_lk_ball_params[:, 0].astype(f32)
    is_h0 = (bt_atom_is_hydrogen == 0)
    ilam = 1.0 / (prm[..., 1] * 2.0 + 1.0)
    tblA = jnp.stack([
        prm[..., 0] * 5.0,
        -_LOG2E * ilam * ilam,
        prm[..., 3] + 1.0,
        (((prm[..., 4] > 0.5) | (prm[..., 7] > 0.5)) & is_h0).astype(f32),
        is_h0.astype(f32),
    ], axis=-1)
    tblN = jnp.tile(jnp.transpose(tblA, (2, 0, 1)), (1, 1, _B))
    pathN = jnp.tile(bt_path_distance.astype(f32), (1, 1, _B))
    msbN = jnp.repeat(pose_stack_min_bond_separation.astype(f32),
                      _A, axis=2)
    torsv = jnp.concatenate([sp2_water_tors, sp3_water_tors]).astype(f32)
    torsc = torsv.reshape(1, _W)
    bt_s = pose_stack_block_type.astype(jnp.int32)
    wdist = water_gen_global_params[:1, :1].astype(f32)
    ramp = lk_ball_global_params[:1, :1].astype(f32)

    out = pl.pallas_call(
        _body,
        grid=(_P, _T),
        compiler_params=pltpu.CompilerParams(
            dimension_semantics=("parallel", "arbitrary")),
        in_specs=[
            pl.BlockSpec((1, 3, _N), lambda p, t: (p, 0, 0)),
            pl.BlockSpec((1, 3, _N), lambda p, t: (p, 0, 0)),
            pl.BlockSpec((5, _NBT, _N), lambda p, t: (0, 0, 0)),
            pl.BlockSpec((_NBT, _A, 5), lambda p, t: (0, 0, 0)),
            pl.BlockSpec((_NBT, _A, _N), lambda p, t: (0, 0, 0)),
            pl.BlockSpec((1, _B, _N), lambda p, t: (p, 0, 0)),
            pl.BlockSpec((1, _W), lambda p, t: (0, 0)),
            pl.BlockSpec(memory_space=pltpu.SMEM),
            pl.BlockSpec(memory_space=pltpu.SMEM),
            pl.BlockSpec(memory_space=pltpu.SMEM),
        ],
        out_specs=pl.BlockSpec((1, 1, 4), lambda p, t: (p, 0, 0)),
        out_shape=jax.ShapeDtypeStruct((_P, 1, 4), f32),
        scratch_shapes=[
            pltpu.VMEM((5 * _W, _N), f32),
            pltpu.VMEM((5, _N), f32),
            pltpu.VMEM((5, _N), f32),
            pltpu.VMEM((5, _N), f32),
            pltpu.VMEM((5, _N), f32),
            pltpu.VMEM((_N, 5), f32),
        ],
    )(crd_r, nb_r, tblN, tblA, pathN, msbN, torsc,
      bt_s, wdist, ramp)
    return out.reshape(_P, 4).T


# speedup vs baseline: 1413.0731x; 1.0257x over previous
import jax
import jax.numpy as jnp
from jax.experimental import pallas as pl
from jax.experimental.pallas import tpu as pltpu

_P, _B, _A, _NBT, _W = 4, 32, 32, 24, 4
_N = _B * _A
_TI = 256
_TB = _TI // _A
_T = _N // _TI
_RX, _RY, _RZ = 0.057, 0.31, 0.95
_LOG2E = 1.4426950408889634

_DN = (((0,), (0,)), ((), ()))


def _body(crdr_ref, nbr_ref, tblN_ref, tblA_ref,
          pathN_ref, msbN_ref, torsc_ref,
          bt_ref, wdist_ref, ramp_ref, out_ref,
          wrc_ref, aug_ref, bocc_ref, bw0_ref, parN_ref, parA_ref):
    f32 = jnp.float32
    pp = pl.program_id(0)
    t = pl.program_id(1)
    r0 = t * _TI
    wd = wdist_ref[0, 0]

    @pl.when(t == 0)
    def _():
        jva = jax.lax.broadcasted_iota(jnp.int32, (1, _N), 1) // _A
        btN = jnp.zeros((1, _N), jnp.int32)
        for b in range(_B):
            btN = jnp.where(jva == b, bt_ref[pp, b], btN)
        for k in range(5):
            acc = jnp.zeros((1, _N), f32)
            for tt in range(_NBT):
                acc = jnp.where(btN == tt, tblN_ref[k, tt:tt + 1, :], acc)
            parN_ref[k:k + 1, :] = acc
        for b in range(_B):
            chunk = tblA_ref[pl.ds(bt_ref[pp, b], 1), :, :]
            parA_ref[b * _A:(b + 1) * _A, :] = chunk[0]
        one = jnp.ones((1, _N), f32)
        cx = crdr_ref[0, 0:1, :]
        cy = crdr_ref[0, 1:2, :]
        cz = crdr_ref[0, 2:3, :]
        csq = cx * cx + cy * cy + cz * cz
        aug_ref[0:3, :] = crdr_ref[0]
        aug_ref[3:4, :] = csq
        aug_ref[4:5, :] = one
        bocc_ref[0:3, :] = crdr_ref[0] * -2.0
        bocc_ref[3:4, :] = one
        bocc_ref[4:5, :] = csq
        nx = nbr_ref[0, 0:1, :]
        ny = nbr_ref[0, 1:2, :]
        nz = nbr_ref[0, 2:3, :]
        vx, vy, vz = cx - nx, cy - ny, cz - nz
        s = 1.0 / (jnp.sqrt(vx * vx + vy * vy + vz * vz) + 1e-6)
        vx, vy, vz = vx * s, vy * s, vz * s
        p1x = vy * _RZ - vz * _RY
        p1y = vz * _RX - vx * _RZ
        p1z = vx * _RY - vy * _RX
        s1 = 1.0 / (jnp.sqrt(p1x * p1x + p1y * p1y + p1z * p1z) + 1e-6)
        p1x, p1y, p1z = p1x * s1, p1y * s1, p1z * s1
        p2x = vy * p1z - vz * p1y
        p2y = vz * p1x - vx * p1z
        p2z = vx * p1y - vy * p1x
        ct = jnp.cos(torsc_ref[...])
        st = jnp.sin(torsc_ref[...])
        for w in range(_W):
            cw = ct[0:1, w:w + 1]
            sw = st[0:1, w:w + 1]
            wx = cx + wd * (p1x * cw + p2x * sw)
            wy = cy + wd * (p1y * cw + p2y * sw)
            wz = cz + wd * (p1z * cw + p2z * sw)
            wrc_ref[5 * w + 0:5 * w + 1, :] = wx
            wrc_ref[5 * w + 1:5 * w + 2, :] = wy
            wrc_ref[5 * w + 2:5 * w + 3, :] = wz
            wrc_ref[5 * w + 3:5 * w + 4, :] = wx * wx + wy * wy + wz * wz
            wrc_ref[5 * w + 4:5 * w + 5, :] = one
        bw0_ref[0:3, :] = wrc_ref[0:3, :] * -2.0
        bw0_ref[3:4, :] = one
        bw0_ref[4:5, :] = wrc_ref[3:4, :]

    intra = jnp.concatenate(
        [pathN_ref[pl.ds(bt_ref[pp, t * _TB + lb], 1), :, :][0]
         for lb in range(_TB)], axis=0)
    inter = jnp.concatenate(
        [jnp.broadcast_to(msbN_ref[0, pl.ds(t * _TB + lb, 1), :], (_A, _N))
         for lb in range(_TB)], axis=0)
    ri = r0 + jax.lax.broadcasted_iota(jnp.int32, (_TI, _N), 0)
    cj = jax.lax.broadcasted_iota(jnp.int32, (_TI, _N), 1)
    same_blk = (ri // _A) == (cj // _A)
    m0 = jnp.where(same_blk, intra, inter)
    hvy_j = parN_ref[4:5, :]
    pol_i = parA_ref[pl.ds(r0, _TI), 3:4]
    pmp = m0 * (pol_i * hvy_j)

    a_at = aug_ref[:, pl.ds(r0, _TI)]
    d2 = jax.lax.dot_general(a_at, bocc_ref[...], _DN,
                             preferred_element_type=f32)
    d = jnp.sqrt(jnp.maximum(d2, 1e-6))
    dg_j = parN_ref[0:1, :]
    rad_j = parN_ref[2:3, :]
    pol_j = parN_ref[3:4, :]
    dg_i = parA_ref[pl.ds(r0, _TI), 0:1]
    nl2_i = parA_ref[pl.ds(r0, _TI), 1:2]
    rad_i = parA_ref[pl.ds(r0, _TI), 2:3]
    dm = d - (rad_i + rad_j)
    lk_iso_p = jnp.exp2(dm * dm * nl2_i) * pmp * dg_i
    s_iso = jnp.sum(lk_iso_p)

    rr = ramp_ref[0, 0]
    c2 = -_LOG2E / (rr * rr)
    occ = jnp.zeros((_TI, _N), f32)
    for w in range(_W):
        a_w = wrc_ref[5 * w:5 * w + 5, pl.ds(r0, _TI)]
        d2w = jax.lax.dot_general(a_w, bocc_ref[...], _DN,
                                  preferred_element_type=f32)
        occ = occ + jnp.exp2(d2w * c2)
    s_ball = jnp.sum(lk_iso_p * jnp.minimum(occ, 1.0))

    a_w0 = wrc_ref[0:5, pl.ds(r0, _TI)]
    d2b = jax.lax.dot_general(a_w0, bw0_ref[...], _DN,
                              preferred_element_type=f32)
    bp = jnp.exp2(d2b * c2) * pmp * pol_j
    s_br = jnp.sum((0.5 * dg_i + 0.5 * dg_j) * bp)
    s_un = jnp.sum(bp)

    i4 = jax.lax.broadcasted_iota(jnp.int32, (1, 4), 1)
    vec = (jnp.where(i4 == 0, s_iso, 0.0) + jnp.where(i4 == 1, s_ball, 0.0)
           + jnp.where(i4 == 2, s_br, 0.0) + jnp.where(i4 == 3, s_un, 0.0))

    @pl.when(t == 0)
    def _():
        out_ref[0] = vec

    @pl.when(t > 0)
    def _():
        out_ref[0] = out_ref[0] + vec


def kernel(pose_coords, pose_stack_block_coord_offset, pose_stack_block_type,
           pose_stack_min_bond_separation, bt_path_distance, bt_atom_is_hydrogen,
           bt_tile_lk_ball_params, lk_ball_global_params, water_gen_global_params,
           sp2_water_tors, sp3_water_tors, ring_water_tors):
    f32 = jnp.float32
    crd_c = pose_coords.astype(f32)
    nb_c = jnp.roll(crd_c, 1, axis=1)
    crd_r = jnp.transpose(crd_c, (0, 2, 1))
    nb_r = jnp.transpose(nb_c, (0, 2, 1))
    prm = bt_tile_lk_ball_params[:, 0].astype(f32)
    is_h0 = (bt_atom_is_hydrogen == 0)
    ilam = 1.0 / (prm[..., 1] * 2.0 + 1.0)
    tblA = jnp.stack([
        prm[..., 0] * 5.0,
        -_LOG2E * ilam * ilam,
        prm[..., 3] + 1.0,
        (((prm[..., 4] > 0.5) | (prm[..., 7] > 0.5)) & is_h0).astype(f32),
        is_h0.astype(f32),
    ], axis=-1)
    tblN = jnp.tile(jnp.transpose(tblA, (2, 0, 1)), (1, 1, _B))
    path01 = ((bt_path_distance >= 4)
              & ~jnp.eye(_A, dtype=bool)[None]).astype(f32)
    pathN = jnp.tile(path01, (1, 1, _B))
    msbN = jnp.repeat((pose_stack_min_bond_separation >= 4).astype(f32),
                      _A, axis=2)
    torsv = jnp.concatenate([sp2_water_tors, sp3_water_tors]).astype(f32)
    torsc = torsv.reshape(1, _W)
    bt_s = pose_stack_block_type.astype(jnp.int32)
    wdist = water_gen_global_params[:1, :1].astype(f32)
    ramp = lk_ball_global_params[:1, :1].astype(f32)

    out = pl.pallas_call(
        _body,
        grid=(_P, _T),
        compiler_params=pltpu.CompilerParams(
            dimension_semantics=("parallel", "arbitrary")),
        in_specs=[
            pl.BlockSpec((1, 3, _N), lambda p, t: (p, 0, 0)),
            pl.BlockSpec((1, 3, _N), lambda p, t: (p, 0, 0)),
            pl.BlockSpec((5, _NBT, _N), lambda p, t: (0, 0, 0)),
            pl.BlockSpec((_NBT, _A, 5), lambda p, t: (0, 0, 0)),
            pl.BlockSpec((_NBT, _A, _N), lambda p, t: (0, 0, 0)),
            pl.BlockSpec((1, _B, _N), lambda p, t: (p, 0, 0)),
            pl.BlockSpec((1, _W), lambda p, t: (0, 0)),
            pl.BlockSpec(memory_space=pltpu.SMEM),
            pl.BlockSpec(memory_space=pltpu.SMEM),
            pl.BlockSpec(memory_space=pltpu.SMEM),
        ],
        out_specs=pl.BlockSpec((1, 1, 4), lambda p, t: (p, 0, 0)),
        out_shape=jax.ShapeDtypeStruct((_P, 1, 4), f32),
        scratch_shapes=[
            pltpu.VMEM((5 * _W, _N), f32),
            pltpu.VMEM((5, _N), f32),
            pltpu.VMEM((5, _N), f32),
            pltpu.VMEM((5, _N), f32),
            pltpu.VMEM((5, _N), f32),
            pltpu.VMEM((_N, 5), f32),
        ],
    )(crd_r, nb_r, tblN, tblA, pathN, msbN, torsc,
      bt_s, wdist, ramp)
    return out.reshape(_P, 4).T


# speedup vs baseline: 1486.5981x; 1.0520x over previous
import jax
import jax.numpy as jnp
from jax.experimental import pallas as pl
from jax.experimental.pallas import tpu as pltpu

_P, _B, _A, _NBT, _W = 4, 32, 32, 24, 4
_N = _B * _A
_TI = 512
_TB = _TI // _A
_T = _N // _TI
_RX, _RY, _RZ = 0.057, 0.31, 0.95
_LOG2E = 1.4426950408889634

_DN = (((0,), (0,)), ((), ()))


def _body(crdr_ref, nbr_ref, tblN_ref, tblA_ref,
          pathN_ref, msbN_ref, torsc_ref,
          bt_ref, wdist_ref, ramp_ref, out_ref,
          wrc_ref, aug_ref, bocc_ref, bw0_ref, parN_ref, parA_ref):
    f32 = jnp.float32
    pp = pl.program_id(0)
    t = pl.program_id(1)
    r0 = t * _TI
    wd = wdist_ref[0, 0]

    @pl.when(t == 0)
    def _():
        jva = jax.lax.broadcasted_iota(jnp.int32, (1, _N), 1) // _A
        btN = jnp.zeros((1, _N), jnp.int32)
        for b in range(_B):
            btN = jnp.where(jva == b, bt_ref[pp, b], btN)
        for k in range(5):
            acc = jnp.zeros((1, _N), f32)
            for tt in range(_NBT):
                acc = jnp.where(btN == tt, tblN_ref[k, tt:tt + 1, :], acc)
            parN_ref[k:k + 1, :] = acc
        for b in range(_B):
            chunk = tblA_ref[pl.ds(bt_ref[pp, b], 1), :, :]
            parA_ref[b * _A:(b + 1) * _A, :] = chunk[0]
        one = jnp.ones((1, _N), f32)
        cx = crdr_ref[0, 0:1, :]
        cy = crdr_ref[0, 1:2, :]
        cz = crdr_ref[0, 2:3, :]
        csq = cx * cx + cy * cy + cz * cz
        aug_ref[0:3, :] = crdr_ref[0]
        aug_ref[3:4, :] = csq
        aug_ref[4:5, :] = one
        bocc_ref[0:3, :] = crdr_ref[0] * -2.0
        bocc_ref[3:4, :] = one
        bocc_ref[4:5, :] = csq
        nx = nbr_ref[0, 0:1, :]
        ny = nbr_ref[0, 1:2, :]
        nz = nbr_ref[0, 2:3, :]
        vx, vy, vz = cx - nx, cy - ny, cz - nz
        s = 1.0 / (jnp.sqrt(vx * vx + vy * vy + vz * vz) + 1e-6)
        vx, vy, vz = vx * s, vy * s, vz * s
        p1x = vy * _RZ - vz * _RY
        p1y = vz * _RX - vx * _RZ
        p1z = vx * _RY - vy * _RX
        s1 = 1.0 / (jnp.sqrt(p1x * p1x + p1y * p1y + p1z * p1z) + 1e-6)
        p1x, p1y, p1z = p1x * s1, p1y * s1, p1z * s1
        p2x = vy * p1z - vz * p1y
        p2y = vz * p1x - vx * p1z
        p2z = vx * p1y - vy * p1x
        ct = jnp.cos(torsc_ref[...])
        st = jnp.sin(torsc_ref[...])
        for w in range(_W):
            cw = ct[0:1, w:w + 1]
            sw = st[0:1, w:w + 1]
            wx = cx + wd * (p1x * cw + p2x * sw)
            wy = cy + wd * (p1y * cw + p2y * sw)
            wz = cz + wd * (p1z * cw + p2z * sw)
            wrc_ref[5 * w + 0:5 * w + 1, :] = wx
            wrc_ref[5 * w + 1:5 * w + 2, :] = wy
            wrc_ref[5 * w + 2:5 * w + 3, :] = wz
            wrc_ref[5 * w + 3:5 * w + 4, :] = wx * wx + wy * wy + wz * wz
            wrc_ref[5 * w + 4:5 * w + 5, :] = one
        bw0_ref[0:3, :] = wrc_ref[0:3, :] * -2.0
        bw0_ref[3:4, :] = one
        bw0_ref[4:5, :] = wrc_ref[3:4, :]

    intra = jnp.concatenate(
        [pathN_ref[pl.ds(bt_ref[pp, t * _TB + lb], 1), :, :][0]
         for lb in range(_TB)], axis=0)
    inter = jnp.concatenate(
        [jnp.broadcast_to(msbN_ref[0, pl.ds(t * _TB + lb, 1), :], (_A, _N))
         for lb in range(_TB)], axis=0)
    ri = r0 + jax.lax.broadcasted_iota(jnp.int32, (_TI, _N), 0)
    cj = jax.lax.broadcasted_iota(jnp.int32, (_TI, _N), 1)
    same_blk = (ri // _A) == (cj // _A)
    m0 = jnp.where(same_blk, intra, inter)
    hvy_j = parN_ref[4:5, :]
    pol_i = parA_ref[pl.ds(r0, _TI), 3:4]
    pmp = m0 * (pol_i * hvy_j)

    a_at = aug_ref[:, pl.ds(r0, _TI)]
    d2 = jax.lax.dot_general(a_at, bocc_ref[...], _DN,
                             preferred_element_type=f32)
    d = jnp.sqrt(jnp.maximum(d2, 1e-6))
    dg_j = parN_ref[0:1, :]
    rad_j = parN_ref[2:3, :]
    pol_j = parN_ref[3:4, :]
    dg_i = parA_ref[pl.ds(r0, _TI), 0:1]
    nl2_i = parA_ref[pl.ds(r0, _TI), 1:2]
    rad_i = parA_ref[pl.ds(r0, _TI), 2:3]
    dm = d - (rad_i + rad_j)
    lk_iso_p = jnp.exp2(dm * dm * nl2_i) * pmp * dg_i
    s_iso = jnp.sum(lk_iso_p)

    rr = ramp_ref[0, 0]
    c2 = -_LOG2E / (rr * rr)
    occ = jnp.zeros((_TI, _N), f32)
    for w in range(_W):
        a_w = wrc_ref[5 * w:5 * w + 5, pl.ds(r0, _TI)]
        d2w = jax.lax.dot_general(a_w, bocc_ref[...], _DN,
                                  preferred_element_type=f32)
        occ = occ + jnp.exp2(d2w * c2)
    s_ball = jnp.sum(lk_iso_p * jnp.minimum(occ, 1.0))

    a_w0 = wrc_ref[0:5, pl.ds(r0, _TI)]
    d2b = jax.lax.dot_general(a_w0, bw0_ref[...], _DN,
                              preferred_element_type=f32)
    bp = jnp.exp2(d2b * c2) * pmp * pol_j
    s_br = jnp.sum((0.5 * dg_i + 0.5 * dg_j) * bp)
    s_un = jnp.sum(bp)

    i4 = jax.lax.broadcasted_iota(jnp.int32, (1, 4), 1)
    vec = (jnp.where(i4 == 0, s_iso, 0.0) + jnp.where(i4 == 1, s_ball, 0.0)
           + jnp.where(i4 == 2, s_br, 0.0) + jnp.where(i4 == 3, s_un, 0.0))

    @pl.when(t == 0)
    def _():
        out_ref[0] = vec

    @pl.when(t > 0)
    def _():
        out_ref[0] = out_ref[0] + vec


def kernel(pose_coords, pose_stack_block_coord_offset, pose_stack_block_type,
           pose_stack_min_bond_separation, bt_path_distance, bt_atom_is_hydrogen,
           bt_tile_lk_ball_params, lk_ball_global_params, water_gen_global_params,
           sp2_water_tors, sp3_water_tors, ring_water_tors):
    f32 = jnp.float32
    crd_c = pose_coords.astype(f32)
    nb_c = jnp.roll(crd_c, 1, axis=1)
    crd_r = jnp.transpose(crd_c, (0, 2, 1))
    nb_r = jnp.transpose(nb_c, (0, 2, 1))
    prm = bt_tile_lk_ball_params[:, 0].astype(f32)
    is_h0 = (bt_atom_is_hydrogen == 0)
    ilam = 1.0 / (prm[..., 1] * 2.0 + 1.0)
    tblA = jnp.stack([
        prm[..., 0] * 5.0,
        -_LOG2E * ilam * ilam,
        prm[..., 3] + 1.0,
        (((prm[..., 4] > 0.5) | (prm[..., 7] > 0.5)) & is_h0).astype(f32),
        is_h0.astype(f32),
    ], axis=-1)
    tblN = jnp.tile(jnp.transpose(tblA, (2, 0, 1)), (1, 1, _B))
    path01 = ((bt_path_distance >= 4)
              & ~jnp.eye(_A, dtype=bool)[None]).astype(f32)
    pathN = jnp.tile(path01, (1, 1, _B))
    msbN = jnp.repeat((pose_stack_min_bond_separation >= 4).astype(f32),
                      _A, axis=2)
    torsv = jnp.concatenate([sp2_water_tors, sp3_water_tors]).astype(f32)
    torsc = torsv.reshape(1, _W)
    bt_s = pose_stack_block_type.astype(jnp.int32)
    wdist = water_gen_global_params[:1, :1].astype(f32)
    ramp = lk_ball_global_params[:1, :1].astype(f32)

    out = pl.pallas_call(
        _body,
        grid=(_P, _T),
        compiler_params=pltpu.CompilerParams(
            dimension_semantics=("parallel", "arbitrary")),
        in_specs=[
            pl.BlockSpec((1, 3, _N), lambda p, t: (p, 0, 0)),
            pl.BlockSpec((1, 3, _N), lambda p, t: (p, 0, 0)),
            pl.BlockSpec((5, _NBT, _N), lambda p, t: (0, 0, 0)),
            pl.BlockSpec((_NBT, _A, 5), lambda p, t: (0, 0, 0)),
            pl.BlockSpec((_NBT, _A, _N), lambda p, t: (0, 0, 0)),
            pl.BlockSpec((1, _B, _N), lambda p, t: (p, 0, 0)),
            pl.BlockSpec((1, _W), lambda p, t: (0, 0)),
            pl.BlockSpec(memory_space=pltpu.SMEM),
            pl.BlockSpec(memory_space=pltpu.SMEM),
            pl.BlockSpec(memory_space=pltpu.SMEM),
        ],
        out_specs=pl.BlockSpec((1, 1, 4), lambda p, t: (p, 0, 0)),
        out_shape=jax.ShapeDtypeStruct((_P, 1, 4), f32),
        scratch_shapes=[
            pltpu.VMEM((5 * _W, _N), f32),
            pltpu.VMEM((5, _N), f32),
            pltpu.VMEM((5, _N), f32),
            pltpu.VMEM((5, _N), f32),
            pltpu.VMEM((5, _N), f32),
            pltpu.VMEM((_N, 5), f32),
        ],
    )(crd_r, nb_r, tblN, tblA, pathN, msbN, torsc,
      bt_s, wdist, ramp)
    return out.reshape(_P, 4).T
